# Initial kernel scaffold; baseline (speedup 1.0000x reference)
import jax
import jax.numpy as jnp
from jax.experimental import pallas as pl


def kernel(som, running_variance, x):
    raise NotImplementedError("write your pallas kernel here")


# speedup vs baseline: 3.7046x; 3.7046x over previous
import jax
import jax.numpy as jnp
from jax import lax
from jax.experimental import pallas as pl
from jax.experimental.pallas import tpu as pltpu

IMG = 64
NU = 64
SHEET = IMG * NU


def _tc_body(xt_ref, som_ref, var_ref, out_ref):
    i = pl.program_id(0)
    d = som_ref[...] - xt_ref[...]
    e = (d * d) / var_ref[...]
    colsum = jnp.sum(e, axis=0, keepdims=True)
    r = lax.broadcasted_iota(jnp.int32, (SHEET, NU), 0) // IMG
    c = lax.broadcasted_iota(jnp.int32, (SHEET, NU), 1)
    g = (r == c).astype(jnp.float32)
    dists = jnp.dot(colsum, g, preferred_element_type=jnp.float32)
    m = jnp.min(dists)

    @pl.when(i == 0)
    def _():
        out_ref[0, 0] = m

    @pl.when(i > 0)
    def _():
        out_ref[0, 0] = jnp.minimum(out_ref[0, 0], m)


@jax.jit
def kernel(som, running_variance, x):
    xt = jnp.tile(x, (1, NU))
    res = pl.pallas_call(
        _tc_body,
        grid=(NU,),
        in_specs=[
            pl.BlockSpec((IMG, SHEET), lambda i: (0, 0)),
            pl.BlockSpec((IMG, SHEET), lambda i: (i, 0)),
            pl.BlockSpec((IMG, SHEET), lambda i: (i, 0)),
        ],
        out_specs=pl.BlockSpec(memory_space=pltpu.SMEM),
        out_shape=jax.ShapeDtypeStruct((1, 1), jnp.float32),
    )(xt, som, running_variance)
    return res[0, 0]


# speedup vs baseline: 4.8897x; 1.3199x over previous
import jax
import jax.numpy as jnp
from jax import lax
from jax.experimental import pallas as pl
from jax.experimental.pallas import tpu as pltpu

IMG = 64
NU = 64
SHEET = IMG * NU
NB = 2


def _tc_body(xt_ref, g_ref, som_ref, var_ref, out_ref):
    i = pl.program_id(0)
    som = som_ref[...].reshape(NB, IMG, SHEET)
    var = var_ref[...].reshape(NB, IMG, SHEET)
    d = som - xt_ref[...][None, :, :]
    e = (d * d) / var
    colsum = jnp.sum(e, axis=1)
    dists = jnp.dot(colsum, g_ref[...], preferred_element_type=jnp.float32)
    m = jnp.min(dists)

    @pl.when(i == 0)
    def _():
        out_ref[0, 0] = m

    @pl.when(i > 0)
    def _():
        out_ref[0, 0] = jnp.minimum(out_ref[0, 0], m)


@jax.jit
def kernel(som, running_variance, x):
    xt = jnp.tile(x, (1, NU))
    r = lax.broadcasted_iota(jnp.int32, (SHEET, NU), 0) // IMG
    c = lax.broadcasted_iota(jnp.int32, (SHEET, NU), 1)
    g = (r == c).astype(jnp.float32)
    res = pl.pallas_call(
        _tc_body,
        grid=(NU // NB,),
        in_specs=[
            pl.BlockSpec((IMG, SHEET), lambda i: (0, 0)),
            pl.BlockSpec((SHEET, NU), lambda i: (0, 0)),
            pl.BlockSpec((NB * IMG, SHEET), lambda i: (i, 0)),
            pl.BlockSpec((NB * IMG, SHEET), lambda i: (i, 0)),
        ],
        out_specs=pl.BlockSpec(memory_space=pltpu.SMEM),
        out_shape=jax.ShapeDtypeStruct((1, 1), jnp.float32),
    )(xt, g, som, running_variance)
    return res[0, 0]


# speedup vs baseline: 5.6220x; 1.1498x over previous
import jax
import jax.numpy as jnp
from jax import lax
from jax.experimental import pallas as pl
from jax.experimental.pallas import tpu as pltpu

IMG = 64
NU = 64
SHEET = IMG * NU
NB = 4


def _tc_body(xt_ref, g_ref, som_ref, var_ref, out_ref):
    i = pl.program_id(0)
    som = som_ref[...].reshape(NB, IMG, SHEET)
    var = var_ref[...].reshape(NB, IMG, SHEET)
    d = som - xt_ref[...][None, :, :]
    e = (d * d) / var
    colsum = jnp.sum(e, axis=1)
    dists = jnp.dot(colsum, g_ref[...], preferred_element_type=jnp.float32)
    m = jnp.min(dists)

    @pl.when(i == 0)
    def _():
        out_ref[0, 0] = m

    @pl.when(i > 0)
    def _():
        out_ref[0, 0] = jnp.minimum(out_ref[0, 0], m)


@jax.jit
def kernel(som, running_variance, x):
    xt = jnp.tile(x, (1, NU))
    r = lax.broadcasted_iota(jnp.int32, (SHEET, NU), 0) // IMG
    c = lax.broadcasted_iota(jnp.int32, (SHEET, NU), 1)
    g = (r == c).astype(jnp.float32)
    res = pl.pallas_call(
        _tc_body,
        grid=(NU // NB,),
        in_specs=[
            pl.BlockSpec((IMG, SHEET), lambda i: (0, 0)),
            pl.BlockSpec((SHEET, NU), lambda i: (0, 0)),
            pl.BlockSpec((NB * IMG, SHEET), lambda i: (i, 0)),
            pl.BlockSpec((NB * IMG, SHEET), lambda i: (i, 0)),
        ],
        out_specs=pl.BlockSpec(memory_space=pltpu.SMEM),
        out_shape=jax.ShapeDtypeStruct((1, 1), jnp.float32),
    )(xt, g, som, running_variance)
    return res[0, 0]
